# baseline (speedup 1.0000x reference)
import functools

import jax
import jax.numpy as jnp
from jax import lax
from jax.experimental import pallas as pl
from jax.experimental.pallas import tpu as pltpu
from jax.experimental.pallas import tpu_sc as plsc

EMBED_DIM = 16
NROWS = 2000000
FIELD_OFFSET = 1000000
NUM_WORKERS = 32
LANE_TILES = NROWS // 128
CHUNK = 512


def _table_phys_flat(table):
    t = table.T.reshape(2, 8, LANE_TILES, 128)
    return t.transpose(0, 2, 1, 3).reshape(-1)


def _sc_gather_t(u_ids, it_ids, tflat):
    B = u_ids.shape[0]
    per_w = B // NUM_WORKERS
    n_planes = 2 * EMBED_DIM
    n_words = per_w * n_planes
    n_chunks = n_words // CHUNK
    chunks_per_plane = per_w // CHUNK
    col_tiles = B // 128
    mesh = plsc.VectorSubcoreMesh(core_axis_name="c", subcore_axis_name="s")

    @functools.partial(
        pl.kernel,
        mesh=mesh,
        out_type=jax.ShapeDtypeStruct((B * n_planes // 128, 128), jnp.float32),
        compiler_params=pltpu.CompilerParams(
            use_tc_tiling_on_sc=False, needs_layout_passes=False),
        scratch_types=[
            pltpu.VMEM((2, per_w), jnp.int32),
            pltpu.VMEM((2, per_w), jnp.int32),
            pltpu.VMEM((n_words,), jnp.int32),
            pltpu.VMEM((n_planes, per_w), jnp.float32),
            pltpu.SemaphoreType.DMA,
        ],
    )
    def k(u_hbm, it_hbm, table_hbm, out_hbm, x_v, rp_v, w_idx, rows_v, sem):
        wid = lax.axis_index("s") * 2 + lax.axis_index("c")
        row0 = wid * per_w
        pltpu.sync_copy(u_hbm.at[pl.ds(row0, per_w)], x_v.at[0, :])
        pltpu.sync_copy(it_hbm.at[pl.ds(row0, per_w)], x_v.at[1, :])

        def build_rp(g, _):
            sl = pl.ds(g * 16, 16)
            for f in range(2):
                r16 = x_v[f, sl] + f * FIELD_OFFSET
                rp_v[f, sl] = ((r16 >> 7) << 10) + (r16 & 127)
            return 0

        lax.fori_loop(0, per_w // 16, build_rp, 0)

        def fire(c, _):
            p = c // chunks_per_plane
            b0 = (c % chunks_per_plane) * CHUNK
            f = p // EMBED_DIM
            j = p % EMBED_DIM
            jconst = (j // 8) * (LANE_TILES * 1024) + (j % 8) * 128
            for g in range(CHUNK // 16):
                sl = pl.ds(b0 + g * 16, 16)
                w_idx[pl.ds(c * CHUNK + g * 16, 16)] = rp_v[f, sl] + jconst
            pltpu.async_copy(
                table_hbm.at[w_idx.at[pl.ds(c * CHUNK, CHUNK)]],
                rows_v.at[p, pl.ds(b0, CHUNK)],
                sem,
            )
            return 0

        lax.fori_loop(0, n_chunks, fire, 0)
        for p in range(n_planes):
            pltpu.make_async_copy(
                table_hbm.at[pl.ds(0, per_w)], rows_v.at[p, :], sem
            ).wait()

        for R in range(n_planes // 8):
            for c in range(per_w // 128):
                t = R * col_tiles + wid * (per_w // 128) + c
                pltpu.sync_copy(
                    rows_v.at[pl.ds(8 * R, 8), pl.ds(128 * c, 128)],
                    out_hbm.at[pl.ds(8 * t, 8), :],
                )

    return k(u_ids, it_ids, tflat)


def _tc_mlp_t(embT, w1t, b1c, w2t, b2c, w3t, b3c, wg, wh, bfc):
    B = embT.shape[1]
    blk = 4096
    grid = (B // blk,)

    def body(e_ref, w1_ref, b1_ref, w2_ref, b2_ref, w3_ref, b3_ref,
             wg_ref, wh_ref, bfc_ref, o_ref):
        e = e_ref[...]
        h = jnp.maximum(
            jnp.dot(w1_ref[...], e, preferred_element_type=jnp.float32)
            + b1_ref[...], 0.0)
        h = jnp.maximum(
            jnp.dot(w2_ref[...], h, preferred_element_type=jnp.float32)
            + b2_ref[...], 0.0)
        h = jnp.maximum(
            jnp.dot(w3_ref[...], h, preferred_element_type=jnp.float32)
            + b3_ref[...], 0.0)
        gmf = e[:EMBED_DIM, :] * e[EMBED_DIM:, :]
        out = (jnp.dot(wg_ref[...], gmf, preferred_element_type=jnp.float32)
               + jnp.dot(wh_ref[...], h, preferred_element_type=jnp.float32)
               + bfc_ref[0])
        o_ref[...] = out[0]

    rep = lambda shape: pl.BlockSpec(shape, lambda i: tuple(0 for _ in shape))
    return pl.pallas_call(
        body,
        grid=grid,
        in_specs=[
            pl.BlockSpec((embT.shape[0], blk), lambda i: (0, i)),
            rep(w1t.shape),
            rep(b1c.shape),
            rep(w2t.shape),
            rep(b2c.shape),
            rep(w3t.shape),
            rep(b3c.shape),
            rep(wg.shape),
            rep(wh.shape),
            rep((1,)),
        ],
        out_specs=pl.BlockSpec((blk,), lambda i: (i,)),
        out_shape=jax.ShapeDtypeStruct((B,), jnp.float32),
    )(embT, w1t, b1c, w2t, b2c, w3t, b3c, wg, wh, bfc)


def kernel(x, table, W1, b1, W2, b2, W3, b3, Wfc, bfc):
    B = x.shape[0]
    x32 = x.astype(jnp.int32)
    out2d = _sc_gather_t(x32[:, 0], x32[:, 1], _table_phys_flat(table))
    embT = (out2d.reshape(4, B // 128, 8, 128)
            .transpose(0, 2, 1, 3)
            .reshape(2 * EMBED_DIM, B))
    return _tc_mlp_t(
        embT,
        W1.T, b1.reshape(-1, 1),
        W2.T, b2.reshape(-1, 1),
        W3.T, b3.reshape(-1, 1),
        Wfc[:EMBED_DIM, :].T, Wfc[EMBED_DIM:, :].T, bfc,
    )


# speedup vs baseline: 1.0121x; 1.0121x over previous
import functools

import jax
import jax.numpy as jnp
from jax import lax
from jax.experimental import pallas as pl
from jax.experimental.pallas import tpu as pltpu
from jax.experimental.pallas import tpu_sc as plsc

EMBED_DIM = 16
NROWS = 2000000
FIELD_OFFSET = 1000000
NUM_WORKERS = 32
LANE_TILES = NROWS // 128
CHUNK = 128


def _table_phys_flat(table):
    t = table.T.reshape(2, 8, LANE_TILES, 128)
    return t.transpose(0, 2, 1, 3).reshape(-1)


def _sc_gather_t(u_ids, it_ids, tflat):
    B = u_ids.shape[0]
    per_w = B // NUM_WORKERS
    n_planes = 2 * EMBED_DIM
    n_words = per_w * n_planes
    n_chunks = n_words // CHUNK
    chunks_per_plane = per_w // CHUNK
    col_tiles = B // 128
    mesh = plsc.VectorSubcoreMesh(core_axis_name="c", subcore_axis_name="s")

    @functools.partial(
        pl.kernel,
        mesh=mesh,
        out_type=jax.ShapeDtypeStruct((B * n_planes // 128, 128), jnp.float32),
        compiler_params=pltpu.CompilerParams(
            use_tc_tiling_on_sc=False, needs_layout_passes=False),
        scratch_types=[
            pltpu.VMEM((2, per_w), jnp.int32),
            pltpu.VMEM((2, per_w), jnp.int32),
            pltpu.VMEM((n_words,), jnp.int32),
            pltpu.VMEM((n_planes, per_w), jnp.float32),
            pltpu.SemaphoreType.DMA,
        ],
    )
    def k(u_hbm, it_hbm, table_hbm, out_hbm, x_v, rp_v, w_idx, rows_v, sem):
        wid = lax.axis_index("s") * 2 + lax.axis_index("c")
        row0 = wid * per_w
        pltpu.sync_copy(u_hbm.at[pl.ds(row0, per_w)], x_v.at[0, :])
        pltpu.sync_copy(it_hbm.at[pl.ds(row0, per_w)], x_v.at[1, :])

        def build_rp(g, _):
            sl = pl.ds(g * 16, 16)
            for f in range(2):
                r16 = x_v[f, sl] + f * FIELD_OFFSET
                rp_v[f, sl] = ((r16 >> 7) << 10) + (r16 & 127)
            return 0

        lax.fori_loop(0, per_w // 16, build_rp, 0)

        def fire(c, _):
            p = c // chunks_per_plane
            b0 = (c % chunks_per_plane) * CHUNK
            f = p // EMBED_DIM
            j = p % EMBED_DIM
            jconst = (j // 8) * (LANE_TILES * 1024) + (j % 8) * 128
            for g in range(CHUNK // 16):
                sl = pl.ds(b0 + g * 16, 16)
                w_idx[pl.ds(c * CHUNK + g * 16, 16)] = rp_v[f, sl] + jconst
            pltpu.async_copy(
                table_hbm.at[w_idx.at[pl.ds(c * CHUNK, CHUNK)]],
                rows_v.at[p, pl.ds(b0, CHUNK)],
                sem,
            )
            return 0

        lax.fori_loop(0, n_chunks, fire, 0)
        for p in range(n_planes):
            pltpu.make_async_copy(
                table_hbm.at[pl.ds(0, per_w)], rows_v.at[p, :], sem
            ).wait()

        for R in range(n_planes // 8):
            for c in range(per_w // 128):
                t = R * col_tiles + wid * (per_w // 128) + c
                pltpu.sync_copy(
                    rows_v.at[pl.ds(8 * R, 8), pl.ds(128 * c, 128)],
                    out_hbm.at[pl.ds(8 * t, 8), :],
                )

    return k(u_ids, it_ids, tflat)


def _tc_mlp_t(embT, w1t, b1c, w2t, b2c, w3t, b3c, wg, wh, bfc):
    B = embT.shape[1]
    blk = 4096
    grid = (B // blk,)

    def body(e_ref, w1_ref, b1_ref, w2_ref, b2_ref, w3_ref, b3_ref,
             wg_ref, wh_ref, bfc_ref, o_ref):
        e = e_ref[...]
        h = jnp.maximum(
            jnp.dot(w1_ref[...], e, preferred_element_type=jnp.float32)
            + b1_ref[...], 0.0)
        h = jnp.maximum(
            jnp.dot(w2_ref[...], h, preferred_element_type=jnp.float32)
            + b2_ref[...], 0.0)
        h = jnp.maximum(
            jnp.dot(w3_ref[...], h, preferred_element_type=jnp.float32)
            + b3_ref[...], 0.0)
        gmf = e[:EMBED_DIM, :] * e[EMBED_DIM:, :]
        out = (jnp.dot(wg_ref[...], gmf, preferred_element_type=jnp.float32)
               + jnp.dot(wh_ref[...], h, preferred_element_type=jnp.float32)
               + bfc_ref[0])
        o_ref[...] = out[0]

    rep = lambda shape: pl.BlockSpec(shape, lambda i: tuple(0 for _ in shape))
    return pl.pallas_call(
        body,
        grid=grid,
        in_specs=[
            pl.BlockSpec((embT.shape[0], blk), lambda i: (0, i)),
            rep(w1t.shape),
            rep(b1c.shape),
            rep(w2t.shape),
            rep(b2c.shape),
            rep(w3t.shape),
            rep(b3c.shape),
            rep(wg.shape),
            rep(wh.shape),
            rep((1,)),
        ],
        out_specs=pl.BlockSpec((blk,), lambda i: (i,)),
        out_shape=jax.ShapeDtypeStruct((B,), jnp.float32),
    )(embT, w1t, b1c, w2t, b2c, w3t, b3c, wg, wh, bfc)


def kernel(x, table, W1, b1, W2, b2, W3, b3, Wfc, bfc):
    B = x.shape[0]
    x32 = x.astype(jnp.int32)
    out2d = _sc_gather_t(x32[:, 0], x32[:, 1], _table_phys_flat(table))
    embT = (out2d.reshape(4, B // 128, 8, 128)
            .transpose(0, 2, 1, 3)
            .reshape(2 * EMBED_DIM, B))
    return _tc_mlp_t(
        embT,
        W1.T, b1.reshape(-1, 1),
        W2.T, b2.reshape(-1, 1),
        W3.T, b3.reshape(-1, 1),
        Wfc[:EMBED_DIM, :].T, Wfc[EMBED_DIM:, :].T, bfc,
    )


# speedup vs baseline: 1.0681x; 1.0553x over previous
import functools

import jax
import jax.numpy as jnp
from jax import lax
from jax.experimental import pallas as pl
from jax.experimental.pallas import tpu as pltpu
from jax.experimental.pallas import tpu_sc as plsc

EMBED_DIM = 16
NROWS = 2000000
FIELD_OFFSET = 1000000
NUM_WORKERS = 32
LANE_TILES = NROWS // 128
CHUNK = 128


def _table_phys_flat(table):
    t = table.T.reshape(2, 8, LANE_TILES, 128)
    return t.transpose(0, 2, 1, 3).reshape(-1)


def _sc_gather_t(u_ids, it_ids, tflat):
    B = u_ids.shape[0]
    per_w = B // NUM_WORKERS
    n_planes = 2 * EMBED_DIM
    n_words = per_w * n_planes
    n_chunks = n_words // CHUNK
    chunks_per_plane = per_w // CHUNK
    col_tiles = B // 128
    mesh = plsc.VectorSubcoreMesh(core_axis_name="c", subcore_axis_name="s")

    @functools.partial(
        pl.kernel,
        mesh=mesh,
        out_type=jax.ShapeDtypeStruct((B * n_planes // 128, 128), jnp.float32),
        compiler_params=pltpu.CompilerParams(
            use_tc_tiling_on_sc=False, needs_layout_passes=False),
        scratch_types=[
            pltpu.VMEM((2, per_w), jnp.int32),
            pltpu.VMEM((2, per_w), jnp.int32),
            pltpu.VMEM((n_words,), jnp.int32),
            pltpu.VMEM((n_planes, per_w), jnp.float32),
            pltpu.VMEM((n_words,), jnp.float32),
            pltpu.SemaphoreType.DMA,
            pltpu.SemaphoreType.DMA,
        ],
    )
    def k(u_hbm, it_hbm, table_hbm, out_hbm, x_v, rp_v, w_idx, rows_v,
          drain_v, sem, osem):
        wid = lax.axis_index("s") * 2 + lax.axis_index("c")
        row0 = wid * per_w
        pltpu.sync_copy(u_hbm.at[pl.ds(row0, per_w)], x_v.at[0, :])
        pltpu.sync_copy(it_hbm.at[pl.ds(row0, per_w)], x_v.at[1, :])

        def build_rp(g, _):
            sl = pl.ds(g * 16, 16)
            for f in range(2):
                r16 = x_v[f, sl] + f * FIELD_OFFSET
                rp_v[f, sl] = ((r16 >> 7) << 10) + (r16 & 127)
            return 0

        lax.fori_loop(0, per_w // 16, build_rp, 0)

        def fire(c, _):
            p = c // chunks_per_plane
            b0 = (c % chunks_per_plane) * CHUNK
            f = p // EMBED_DIM
            j = p % EMBED_DIM
            jconst = (j // 8) * (LANE_TILES * 1024) + (j % 8) * 128
            for g in range(CHUNK // 16):
                sl = pl.ds(b0 + g * 16, 16)
                w_idx[pl.ds(c * CHUNK + g * 16, 16)] = rp_v[f, sl] + jconst
            pltpu.async_copy(
                table_hbm.at[w_idx.at[pl.ds(c * CHUNK, CHUNK)]],
                rows_v.at[p, pl.ds(b0, CHUNK)],
                sem,
            )
            return 0

        lax.fori_loop(0, n_chunks, fire, 0)
        pltpu.make_async_copy(
            table_hbm.at[pl.ds(0, n_words)], drain_v, sem
        ).wait()

        out_copies = []
        for R in range(n_planes // 8):
            for c in range(per_w // 128):
                t = R * col_tiles + wid * (per_w // 128) + c
                out_copies.append(pltpu.async_copy(
                    rows_v.at[pl.ds(8 * R, 8), pl.ds(128 * c, 128)],
                    out_hbm.at[pl.ds(8 * t, 8), :],
                    osem,
                ))
        for cpy in out_copies:
            cpy.wait()

    return k(u_ids, it_ids, tflat)


def _tc_mlp_t(embT, w1t, b1c, w2t, b2c, w3t, b3c, wg, wh, bfc):
    B = embT.shape[1]
    blk = B
    grid = (B // blk,)

    def body(e_ref, w1_ref, b1_ref, w2_ref, b2_ref, w3_ref, b3_ref,
             wg_ref, wh_ref, bfc_ref, o_ref):
        e = e_ref[...]
        h = jnp.maximum(
            jnp.dot(w1_ref[...], e, preferred_element_type=jnp.float32)
            + b1_ref[...], 0.0)
        h = jnp.maximum(
            jnp.dot(w2_ref[...], h, preferred_element_type=jnp.float32)
            + b2_ref[...], 0.0)
        h = jnp.maximum(
            jnp.dot(w3_ref[...], h, preferred_element_type=jnp.float32)
            + b3_ref[...], 0.0)
        gmf = e[:EMBED_DIM, :] * e[EMBED_DIM:, :]
        out = (jnp.dot(wg_ref[...], gmf, preferred_element_type=jnp.float32)
               + jnp.dot(wh_ref[...], h, preferred_element_type=jnp.float32)
               + bfc_ref[0])
        o_ref[...] = out[0]

    rep = lambda shape: pl.BlockSpec(shape, lambda i: tuple(0 for _ in shape))
    return pl.pallas_call(
        body,
        grid=grid,
        in_specs=[
            pl.BlockSpec((embT.shape[0], blk), lambda i: (0, i)),
            rep(w1t.shape),
            rep(b1c.shape),
            rep(w2t.shape),
            rep(b2c.shape),
            rep(w3t.shape),
            rep(b3c.shape),
            rep(wg.shape),
            rep(wh.shape),
            rep((1,)),
        ],
        out_specs=pl.BlockSpec((blk,), lambda i: (i,)),
        out_shape=jax.ShapeDtypeStruct((B,), jnp.float32),
    )(embT, w1t, b1c, w2t, b2c, w3t, b3c, wg, wh, bfc)


def kernel(x, table, W1, b1, W2, b2, W3, b3, Wfc, bfc):
    B = x.shape[0]
    x32 = x.astype(jnp.int32)
    out2d = _sc_gather_t(x32[:, 0], x32[:, 1], _table_phys_flat(table))
    embT = (out2d.reshape(4, B // 128, 8, 128)
            .transpose(0, 2, 1, 3)
            .reshape(2 * EMBED_DIM, B))
    return _tc_mlp_t(
        embT,
        W1.T, b1.reshape(-1, 1),
        W2.T, b2.reshape(-1, 1),
        W3.T, b3.reshape(-1, 1),
        Wfc[:EMBED_DIM, :].T, Wfc[EMBED_DIM:, :].T, bfc,
    )
